# baseline (speedup 1.0000x reference)
import jax
import jax.numpy as jnp
from jax.experimental import pallas as pl
from jax.experimental.pallas import tpu as pltpu

B, T, D, K = 16, 2048, 128, 512
BLK = 8192
NBLK = (B * T) // BLK
TPB = T // BLK


def _vq_block(lens_ref, x_ref, cb_ref, cnorm_ref, q_ref, idx_ref, loss_ref):
    pid = pl.program_id(0)
    x = x_ref[0]
    cb = cb_ref[...]
    scores = jax.lax.dot_general(
        x.astype(jnp.bfloat16), cb.astype(jnp.bfloat16),
        (((1,), (1,)), ((), ())),
        preferred_element_type=jnp.float32)
    xnorm = jnp.sum(x * x, axis=1, keepdims=True)
    d2 = (xnorm - 2.0 * scores) + cnorm_ref[...]
    minv = jnp.min(d2, axis=1, keepdims=True)
    iota_k = jax.lax.broadcasted_iota(jnp.int32, (BLK, K), 1)
    idx = jnp.min(jnp.where(d2 == minv, iota_k, K),
                  axis=1, keepdims=True)
    onehot = (iota_k == idx).astype(jnp.bfloat16)
    q = jax.lax.dot_general(
        onehot, cb.astype(jnp.bfloat16), (((1,), (0,)), ((), ())),
        preferred_element_type=jnp.float32)

    iota_r = jax.lax.broadcasted_iota(jnp.int32, (BLK, 1), 0)
    if BLK >= T:
        MPB = BLK // T
        tpos = iota_r & (T - 1)
        seg = iota_r >> (T.bit_length() - 1)
        blen = lens_ref[pid * MPB]
        for j in range(1, MPB):
            blen = jnp.where(seg >= j, lens_ref[pid * MPB + j], blen)
    else:
        b = pid // TPB
        tpos = (pid % TPB) * BLK + iota_r
        blen = lens_ref[b]
    mask = tpos < blen

    q_ref[0] = jnp.where(mask, q, x)
    idx_ref[0] = jnp.where(mask, idx, -1)

    diff = q - x
    part = jnp.sum(jnp.where(mask, diff * diff, 0.0))

    @pl.when(pid == 0)
    def _init():
        loss_ref[0, 0] = 0.0

    loss_ref[0, 0] += part


@jax.jit
def kernel(segmented_feats, segmented_feats_lengths, codebook):
    xf = segmented_feats.reshape(NBLK, BLK, D)
    cnorm = jnp.sum(codebook * codebook, axis=1)[None, :]
    grid_spec = pltpu.PrefetchScalarGridSpec(
        num_scalar_prefetch=1,
        grid=(NBLK,),
        in_specs=[
            pl.BlockSpec((1, BLK, D), lambda i, lens: (i, 0, 0)),
            pl.BlockSpec((K, D), lambda i, lens: (0, 0)),
            pl.BlockSpec((1, K), lambda i, lens: (0, 0)),
        ],
        out_specs=[
            pl.BlockSpec((1, BLK, D), lambda i, lens: (i, 0, 0)),
            pl.BlockSpec((1, BLK, 1), lambda i, lens: (i, 0, 0)),
            pl.BlockSpec((1, 1), lambda i, lens: (0, 0),
                         memory_space=pltpu.SMEM),
        ],
    )
    q, idx, loss = pl.pallas_call(
        _vq_block,
        grid_spec=grid_spec,
        out_shape=[
            jax.ShapeDtypeStruct((NBLK, BLK, D), jnp.float32),
            jax.ShapeDtypeStruct((NBLK, BLK, 1), jnp.int32),
            jax.ShapeDtypeStruct((1, 1), jnp.float32),
        ],
    )(segmented_feats_lengths, xf, codebook, cnorm)

    quantized_out = q.reshape(B, T, D)
    indices_out = idx.reshape(B, T)
    denom = jnp.maximum(
        jnp.sum(segmented_feats_lengths).astype(jnp.float32) * D, 1.0)
    commit_loss = loss[0, 0] / denom
    return quantized_out, indices_out, commit_loss


# speedup vs baseline: 1.0428x; 1.0428x over previous
import jax
import jax.numpy as jnp
from jax.experimental import pallas as pl
from jax.experimental.pallas import tpu as pltpu

B, T, D, K = 16, 2048, 128, 512
BLK = 4096
NBLK = (B * T) // BLK
TPB = T // BLK


def _vq_block(lens_ref, x_ref, cb_ref, cnorm_ref, q_ref, idx_ref, loss_ref):
    pid = pl.program_id(0)
    x = x_ref[0]
    cb = cb_ref[...]
    scores = jax.lax.dot_general(
        x.astype(jnp.bfloat16), cb.astype(jnp.bfloat16),
        (((1,), (1,)), ((), ())),
        preferred_element_type=jnp.float32)
    xnorm = jnp.sum(x * x, axis=1, keepdims=True)
    d2 = (xnorm - 2.0 * scores) + cnorm_ref[...]
    idx = jnp.argmin(d2, axis=1, keepdims=True).astype(jnp.int32)
    iota_k = jax.lax.broadcasted_iota(jnp.int32, (BLK, K), 1)
    onehot = (iota_k == idx).astype(jnp.bfloat16)
    q = jax.lax.dot_general(
        onehot, cb.astype(jnp.bfloat16), (((1,), (0,)), ((), ())),
        preferred_element_type=jnp.float32)

    iota_r = jax.lax.broadcasted_iota(jnp.int32, (BLK, 1), 0)
    if BLK >= T:
        MPB = BLK // T
        tpos = iota_r & (T - 1)
        seg = iota_r >> (T.bit_length() - 1)
        blen = lens_ref[pid * MPB]
        for j in range(1, MPB):
            blen = jnp.where(seg >= j, lens_ref[pid * MPB + j], blen)
    else:
        b = pid // TPB
        tpos = (pid % TPB) * BLK + iota_r
        blen = lens_ref[b]
    mask = tpos < blen

    q_ref[0] = jnp.where(mask, q, x)
    idx_ref[0] = jnp.where(mask, idx, -1)

    diff = q - x
    part = jnp.sum(jnp.where(mask, diff * diff, 0.0))

    @pl.when(pid == 0)
    def _init():
        loss_ref[0, 0] = 0.0

    loss_ref[0, 0] += part


@jax.jit
def kernel(segmented_feats, segmented_feats_lengths, codebook):
    xf = segmented_feats.reshape(NBLK, BLK, D)
    cnorm = jnp.sum(codebook * codebook, axis=1)[None, :]
    grid_spec = pltpu.PrefetchScalarGridSpec(
        num_scalar_prefetch=1,
        grid=(NBLK,),
        in_specs=[
            pl.BlockSpec((1, BLK, D), lambda i, lens: (i, 0, 0)),
            pl.BlockSpec((K, D), lambda i, lens: (0, 0)),
            pl.BlockSpec((1, K), lambda i, lens: (0, 0)),
        ],
        out_specs=[
            pl.BlockSpec((1, BLK, D), lambda i, lens: (i, 0, 0)),
            pl.BlockSpec((1, BLK, 1), lambda i, lens: (i, 0, 0)),
            pl.BlockSpec((1, 1), lambda i, lens: (0, 0),
                         memory_space=pltpu.SMEM),
        ],
    )
    q, idx, loss = pl.pallas_call(
        _vq_block,
        grid_spec=grid_spec,
        out_shape=[
            jax.ShapeDtypeStruct((NBLK, BLK, D), jnp.float32),
            jax.ShapeDtypeStruct((NBLK, BLK, 1), jnp.int32),
            jax.ShapeDtypeStruct((1, 1), jnp.float32),
        ],
    )(segmented_feats_lengths, xf, codebook, cnorm)

    quantized_out = q.reshape(B, T, D)
    indices_out = idx.reshape(B, T)
    denom = jnp.maximum(
        jnp.sum(segmented_feats_lengths).astype(jnp.float32) * D, 1.0)
    commit_loss = loss[0, 0] / denom
    return quantized_out, indices_out, commit_loss


# speedup vs baseline: 1.0470x; 1.0040x over previous
import jax
import jax.numpy as jnp
from jax.experimental import pallas as pl
from jax.experimental.pallas import tpu as pltpu

B, T, D, K = 16, 2048, 128, 512
BLK = 4096
NBLK = (B * T) // BLK
TPB = T // BLK


def _vq_block(lens_ref, x_ref, cb_ref, cnorm_ref, q_ref, idx_ref, loss_ref):
    pid = pl.program_id(0)
    x = x_ref[0]
    cb = cb_ref[...]
    m2s = jax.lax.dot_general(
        (-2.0 * x).astype(jnp.bfloat16), cb.astype(jnp.bfloat16),
        (((1,), (1,)), ((), ())),
        preferred_element_type=jnp.float32)
    d2 = m2s + cnorm_ref[...]
    idx = jnp.argmin(d2, axis=1, keepdims=True).astype(jnp.int32)
    iota_k = jax.lax.broadcasted_iota(jnp.int32, (BLK, K), 1)
    onehot = (iota_k == idx).astype(jnp.bfloat16)
    q = jax.lax.dot_general(
        onehot, cb.astype(jnp.bfloat16), (((1,), (0,)), ((), ())),
        preferred_element_type=jnp.float32)

    iota_r = jax.lax.broadcasted_iota(jnp.int32, (BLK, 1), 0)
    if BLK >= T:
        MPB = BLK // T
        tpos = iota_r & (T - 1)
        seg = iota_r >> (T.bit_length() - 1)
        blen = lens_ref[pid * MPB]
        for j in range(1, MPB):
            blen = jnp.where(seg >= j, lens_ref[pid * MPB + j], blen)
    else:
        b = pid // TPB
        tpos = (pid % TPB) * BLK + iota_r
        blen = lens_ref[b]
    mask = tpos < blen

    q_ref[0] = jnp.where(mask, q, x)
    idx_ref[0] = jnp.where(mask, idx, -1)

    diff = q - x
    part = jnp.sum(jnp.where(mask, diff * diff, 0.0))

    @pl.when(pid == 0)
    def _init():
        loss_ref[0, 0] = 0.0

    loss_ref[0, 0] += part


@jax.jit
def kernel(segmented_feats, segmented_feats_lengths, codebook):
    xf = segmented_feats.reshape(NBLK, BLK, D)
    cnorm = jnp.sum(codebook * codebook, axis=1)[None, :]
    grid_spec = pltpu.PrefetchScalarGridSpec(
        num_scalar_prefetch=1,
        grid=(NBLK,),
        in_specs=[
            pl.BlockSpec((1, BLK, D), lambda i, lens: (i, 0, 0)),
            pl.BlockSpec((K, D), lambda i, lens: (0, 0)),
            pl.BlockSpec((1, K), lambda i, lens: (0, 0)),
        ],
        out_specs=[
            pl.BlockSpec((1, BLK, D), lambda i, lens: (i, 0, 0)),
            pl.BlockSpec((1, BLK, 1), lambda i, lens: (i, 0, 0)),
            pl.BlockSpec((1, 1), lambda i, lens: (0, 0),
                         memory_space=pltpu.SMEM),
        ],
    )
    q, idx, loss = pl.pallas_call(
        _vq_block,
        grid_spec=grid_spec,
        out_shape=[
            jax.ShapeDtypeStruct((NBLK, BLK, D), jnp.float32),
            jax.ShapeDtypeStruct((NBLK, BLK, 1), jnp.int32),
            jax.ShapeDtypeStruct((1, 1), jnp.float32),
        ],
    )(segmented_feats_lengths, xf, codebook, cnorm)

    quantized_out = q.reshape(B, T, D)
    indices_out = idx.reshape(B, T)
    denom = jnp.maximum(
        jnp.sum(segmented_feats_lengths).astype(jnp.float32) * D, 1.0)
    commit_loss = loss[0, 0] / denom
    return quantized_out, indices_out, commit_loss


# speedup vs baseline: 1.0516x; 1.0044x over previous
import jax
import jax.numpy as jnp
from jax.experimental import pallas as pl
from jax.experimental.pallas import tpu as pltpu

B, T, D, K = 16, 2048, 128, 512
MPB = 2
BLK = MPB * T
NBLK = B // MPB


def _vq_block(lens_ref, x_ref, cb_ref, cnorm_ref, q_ref, idx_ref, loss_ref):
    pid = pl.program_id(0)
    x = x_ref[...].reshape(BLK, D)
    cb = cb_ref[...]
    m2s = jax.lax.dot_general(
        (-2.0 * x).astype(jnp.bfloat16), cb.astype(jnp.bfloat16),
        (((1,), (1,)), ((), ())),
        preferred_element_type=jnp.float32)
    d2 = m2s + cnorm_ref[...]
    idx = jnp.argmin(d2, axis=1, keepdims=True).astype(jnp.int32)
    iota_k = jax.lax.broadcasted_iota(jnp.int32, (BLK, K), 1)
    onehot = (iota_k == idx).astype(jnp.bfloat16)
    q = jax.lax.dot_general(
        onehot, cb.astype(jnp.bfloat16), (((1,), (0,)), ((), ())),
        preferred_element_type=jnp.float32)

    iota_r = jax.lax.broadcasted_iota(jnp.int32, (BLK, 1), 0)
    tpos = iota_r & (T - 1)
    seg = iota_r >> (T.bit_length() - 1)
    blen = lens_ref[pid * MPB]
    for j in range(1, MPB):
        blen = jnp.where(seg >= j, lens_ref[pid * MPB + j], blen)
    mask = tpos < blen

    q_ref[...] = jnp.where(mask, q, x).reshape(MPB, T, D)
    idx_ref[...] = jnp.where(mask, idx, -1).reshape(MPB, T, 1)

    diff = q - x
    part = jnp.sum(jnp.where(mask, diff * diff, 0.0))

    @pl.when(pid == 0)
    def _init():
        loss_ref[0, 0] = 0.0

    loss_ref[0, 0] += part


@jax.jit
def kernel(segmented_feats, segmented_feats_lengths, codebook):
    cnorm = jnp.sum(codebook * codebook, axis=1)[None, :]
    grid_spec = pltpu.PrefetchScalarGridSpec(
        num_scalar_prefetch=1,
        grid=(NBLK,),
        in_specs=[
            pl.BlockSpec((MPB, T, D), lambda i, lens: (i, 0, 0)),
            pl.BlockSpec((K, D), lambda i, lens: (0, 0)),
            pl.BlockSpec((1, K), lambda i, lens: (0, 0)),
        ],
        out_specs=[
            pl.BlockSpec((MPB, T, D), lambda i, lens: (i, 0, 0)),
            pl.BlockSpec((MPB, T, 1), lambda i, lens: (i, 0, 0)),
            pl.BlockSpec((1, 1), lambda i, lens: (0, 0),
                         memory_space=pltpu.SMEM),
        ],
    )
    q, idx, loss = pl.pallas_call(
        _vq_block,
        grid_spec=grid_spec,
        out_shape=[
            jax.ShapeDtypeStruct((B, T, D), jnp.float32),
            jax.ShapeDtypeStruct((B, T, 1), jnp.int32),
            jax.ShapeDtypeStruct((1, 1), jnp.float32),
        ],
    )(segmented_feats_lengths, segmented_feats, codebook, cnorm)

    quantized_out = q
    indices_out = idx.reshape(B, T)
    denom = jnp.maximum(
        jnp.sum(segmented_feats_lengths).astype(jnp.float32) * D, 1.0)
    commit_loss = loss[0, 0] / denom
    return quantized_out, indices_out, commit_loss
